# baseline (speedup 1.0000x reference)
import functools

import jax
import jax.numpy as jnp
from jax import lax
from jax.experimental import pallas as pl
from jax.experimental.pallas import tpu as pltpu
from jax.experimental.pallas import tpu_sc as plsc

N_NODES = 10000
N_EDGES = 320000
D_FEAT = 128
D_HALF = D_FEAT // 2

NUM_CORES = 2
NUM_SUBCORES = 16
NW = NUM_CORES * NUM_SUBCORES
EPT = N_EDGES // NUM_SUBCORES
CHUNK = 80
NCHUNK = EPT // CHUNK
NB = 2
L = 16
UNROLL = 8


def _partial_dots(z1a, z1b, z2a, z2b, ei_hbm, pout,
                  sh1, sh2, sidx, didx, arows, brows, ov,
                  ga, gb, so, gi1, gi2):
    c_ax = lax.axis_index("c")
    s_ax = lax.axis_index("s")
    base_e = s_ax * EPT
    base_o = c_ax * N_EDGES + base_e

    @pl.when(jnp.logical_and(s_ax == 0, c_ax == 0))
    def _():
        pltpu.sync_copy(z1a, sh1)
        pltpu.sync_copy(z2a, sh2)

    @pl.when(jnp.logical_and(s_ax == 0, c_ax == 1))
    def _():
        pltpu.sync_copy(z1b, sh1)
        pltpu.sync_copy(z2b, sh2)

    plsc.subcore_barrier()

    NI = 2 * NB

    def issue_idx(cc, j):
        pltpu.async_copy(ei_hbm.at[pl.ds(base_e + cc * CHUNK, CHUNK)],
                         sidx[j], gi1[j])
        pltpu.async_copy(
            ei_hbm.at[pl.ds(N_EDGES + base_e + cc * CHUNK, CHUNK)],
            didx[j], gi2[j])

    def wait_idx(j):
        pltpu.make_async_copy(ei_hbm.at[pl.ds(0, CHUNK)], sidx[j],
                              gi1[j]).wait()
        pltpu.make_async_copy(ei_hbm.at[pl.ds(0, CHUNK)], didx[j],
                              gi2[j]).wait()

    def issue_gather(j, b):
        pltpu.async_copy(sh1.at[sidx[j]], arows[b], ga[b])
        pltpu.async_copy(sh2.at[didx[j]], brows[b], gb[b])

    def wait_gather(b):
        pltpu.make_async_copy(z1a.at[pl.ds(0, CHUNK)], arows[b],
                              ga[b]).wait()
        pltpu.make_async_copy(z2a.at[pl.ds(0, CHUNK)], brows[b],
                              gb[b]).wait()

    def wait_out(b):
        pltpu.make_async_copy(ov[b], pout.at[pl.ds(base_o, CHUNK)],
                              so[b]).wait()

    def compute(cc, b):
        def g_body(g, carry):
            rows = lax.iota(jnp.int32, L) + g * L

            def d_body(d, acc):
                for u in range(UNROLL):
                    cols = jnp.full((L,), d * UNROLL + u, jnp.int32)
                    va = plsc.load_gather(arows[b], [rows, cols])
                    vb = plsc.load_gather(brows[b], [rows, cols])
                    acc = acc + va * vb
                return acc

            acc = lax.fori_loop(0, D_HALF // UNROLL, d_body,
                                jnp.zeros((L,), jnp.float32))
            ov[b][pl.ds(g * L, L)] = acc
            return carry

        lax.fori_loop(0, CHUNK // L, g_body, 0)
        pltpu.async_copy(ov[b],
                         pout.at[pl.ds(base_o + cc * CHUNK, CHUNK)],
                         so[b])

    issue_idx(jnp.int32(2), 0)
    issue_idx(jnp.int32(3), 1)
    issue_idx(jnp.int32(0), 2)
    issue_idx(jnp.int32(1), 3)
    wait_idx(2)
    wait_idx(3)
    issue_gather(2, 0)
    issue_gather(3, 1)

    n_main = (NCHUNK - 2) // NI

    def turn(t, carry):
        for k in range(NI):
            b = k % NB
            cc = t * NI + k
            wait_gather(b)
            issue_idx(jnp.minimum(cc + NI, NCHUNK - 1), (k + 2) % NI)

            @pl.when(jnp.logical_or(t > 0, k >= NB))
            def _():
                wait_out(b)

            compute(cc, b)
            wait_idx(k)
            issue_gather(k, b)
        return carry

    lax.fori_loop(0, n_main, turn, 0)

    for b in range(NB):
        wait_gather(b)
        wait_out(b)
        compute(n_main * NI + b, b)
        wait_idx(b)
    for b in range(NB):
        wait_out(b)


def _combine(pout, out_hbm, v1, v2, vo):
    c_ax = lax.axis_index("c")
    s_ax = lax.axis_index("s")
    wid = s_ax * NUM_CORES + c_ax
    base = wid * (N_EDGES // NW)

    pltpu.sync_copy(pout.at[pl.ds(base, N_EDGES // NW)], v1)
    pltpu.sync_copy(pout.at[pl.ds(N_EDGES + base, N_EDGES // NW)], v2)

    def body(i, carry):
        v = v1[pl.ds(i * L, L)] + v2[pl.ds(i * L, L)]
        vo[pl.ds(i * L, L)] = 1.0 / (1.0 + jnp.exp(-v))
        return carry

    lax.fori_loop(0, (N_EDGES // NW) // L, body, 0)
    pltpu.sync_copy(vo, out_hbm.at[pl.ds(base, N_EDGES // NW)])


def kernel(z1, z2, edge_index):
    ei = edge_index.astype(jnp.int32).reshape(-1)
    z1a = z1[:, :D_HALF]
    z1b = z1[:, D_HALF:]
    z2a = z2[:, :D_HALF]
    z2b = z2[:, D_HALF:]
    mesh = plsc.VectorSubcoreMesh(core_axis_name="c", subcore_axis_name="s")
    params = pltpu.CompilerParams(needs_layout_passes=False)

    partials = functools.partial(
        pl.kernel,
        mesh=mesh,
        compiler_params=pltpu.CompilerParams(
            needs_layout_passes=False, use_tc_tiling_on_sc=False),
        out_type=jax.ShapeDtypeStruct((2 * N_EDGES,), jnp.float32),
        scratch_types=[
            pltpu.VMEM_SHARED((N_NODES, D_HALF), jnp.float32),
            pltpu.VMEM_SHARED((N_NODES, D_HALF), jnp.float32),
            [pltpu.VMEM((CHUNK,), jnp.int32) for _ in range(2 * NB)],
            [pltpu.VMEM((CHUNK,), jnp.int32) for _ in range(2 * NB)],
            [pltpu.VMEM((CHUNK, D_HALF), jnp.float32) for _ in range(NB)],
            [pltpu.VMEM((CHUNK, D_HALF), jnp.float32) for _ in range(NB)],
            [pltpu.VMEM((CHUNK,), jnp.float32) for _ in range(NB)],
            [pltpu.SemaphoreType.DMA for _ in range(NB)],
            [pltpu.SemaphoreType.DMA for _ in range(NB)],
            [pltpu.SemaphoreType.DMA for _ in range(NB)],
            [pltpu.SemaphoreType.DMA for _ in range(2 * NB)],
            [pltpu.SemaphoreType.DMA for _ in range(2 * NB)],
        ],
    )(_partial_dots)
    p = partials(z1a, z1b, z2a, z2b, ei)

    combine = functools.partial(
        pl.kernel,
        mesh=mesh,
        compiler_params=params,
        out_type=jax.ShapeDtypeStruct((N_EDGES,), jnp.float32),
        scratch_types=[
            pltpu.VMEM((N_EDGES // NW,), jnp.float32),
            pltpu.VMEM((N_EDGES // NW,), jnp.float32),
            pltpu.VMEM((N_EDGES // NW,), jnp.float32),
        ],
    )(_combine)
    return combine(p)
